# baseline (speedup 1.0000x reference)
import functools

import jax
import jax.numpy as jnp
from jax import lax
from jax.experimental import pallas as pl
from jax.experimental.pallas import tpu as pltpu
from jax.experimental.pallas import tpu_sc as plsc

C = 128
N = 10000
E = 320000
NC = 2
NS = 16
NW = NC * NS
K = 128
CW = 80
EW = K * CW
E_PAD = NW * EW
N_ACC = 10240
RPT = N_ACC // NS
BLK = 2048


@functools.lru_cache(maxsize=None)
def _mesh():
    return plsc.VectorSubcoreMesh(
        core_axis_name="c", subcore_axis_name="s",
        num_cores=NC, num_subcores=NS)


NBUF = 3


def _gather_body(x_hbm, src_hbm, dst_hbm, xs_hbm, xr_hbm,
                 idx_s, idx_r, *bufs_and_sems):
    rs = bufs_and_sems[0:NBUF]
    rr = bufs_and_sems[NBUF:2 * NBUF]
    ss = bufs_and_sems[2 * NBUF:3 * NBUF]
    sr = bufs_and_sems[3 * NBUF:4 * NBUF]
    cid = lax.axis_index("c")
    sid = lax.axis_index("s")
    wid = sid * NC + cid
    pltpu.sync_copy(src_hbm.at[wid], idx_s)
    pltpu.sync_copy(dst_hbm.at[wid], idx_r)
    base = wid * EW

    def fire(c, b):
        pltpu.async_copy(x_hbm.at[idx_s.at[c]], rs[b], ss[b])
        pltpu.async_copy(x_hbm.at[idx_r.at[c]], rr[b], sr[b])

    def waitpair(b):
        pltpu.make_async_copy(x_hbm.at[idx_s.at[0]], rs[b], ss[b]).wait()
        pltpu.make_async_copy(x_hbm.at[idx_r.at[0]], rr[b], sr[b]).wait()

    def writeout(c, b):
        off = pl.multiple_of(base + c * K, 8)
        pltpu.sync_copy(rs[b], xs_hbm.at[pl.ds(off, K)])
        pltpu.sync_copy(rr[b], xr_hbm.at[pl.ds(off, K)])

    for b in range(NBUF):
        fire(b, b)

    nsteps = (CW - NBUF) // NBUF

    def step(i, carry):
        c0 = NBUF * i
        for b in range(NBUF):
            waitpair(b)
            writeout(c0 + b, b)
            fire(c0 + b + NBUF, b)
        return carry

    lax.fori_loop(0, nsteps, step, 0)
    done = nsteps * NBUF
    for c in range(done, CW):
        b = c % NBUF
        waitpair(b)
        writeout(c, b)
        if c + NBUF < CW:
            fire(c + NBUF, b)


def _gather(xb, src_w, dst_w):
    fn = pl.kernel(
        _gather_body,
        out_type=(jax.ShapeDtypeStruct((E_PAD, C), jnp.float32),
                  jax.ShapeDtypeStruct((E_PAD, C), jnp.float32)),
        mesh=_mesh(),
        scratch_types=(
            [pltpu.VMEM((CW, K), jnp.int32)] * 2
            + [pltpu.VMEM((K, C), jnp.float32)] * (2 * NBUF)
            + [pltpu.SemaphoreType.DMA] * (2 * NBUF)
        ),
    )
    return fn(xb, src_w, dst_w)


def _scatter_body(flux_hbm, dst_hbm, zeros_hbm, parts_hbm,
                  idx_v, fb0, fb1, acc, sl0, sl1):
    cid = lax.axis_index("c")
    sid = lax.axis_index("s")
    wid = sid * NC + cid
    pltpu.sync_copy(zeros_hbm.at[pl.ds(sid * RPT, RPT)],
                    acc.at[pl.ds(sid * RPT, RPT)])
    pltpu.sync_copy(dst_hbm.at[wid], idx_v)
    plsc.subcore_barrier()
    base = wid * EW

    def fire(c, fb, sl):
        off = pl.multiple_of(base + c * K, 8)
        pltpu.async_copy(flux_hbm.at[pl.ds(off, K)], fb, sl)

    def wait(fb, sl):
        pltpu.make_async_copy(flux_hbm.at[pl.ds(0, K)], fb, sl).wait()

    def scat(c, fb):
        pltpu.sync_copy(fb, acc.at[idx_v.at[c]], add=True)

    fire(0, fb0, sl0)
    fire(1, fb1, sl1)

    def step(i, carry):
        c = 2 * i
        wait(fb0, sl0)
        scat(c, fb0)
        fire(c + 2, fb0, sl0)
        wait(fb1, sl1)
        scat(c + 1, fb1)
        fire(c + 3, fb1, sl1)
        return carry

    lax.fori_loop(0, (CW - 2) // 2, step, 0)
    wait(fb0, sl0)
    scat(CW - 2, fb0)
    wait(fb1, sl1)
    scat(CW - 1, fb1)

    plsc.subcore_barrier()
    pltpu.sync_copy(acc.at[pl.ds(sid * RPT, RPT)],
                    parts_hbm.at[cid, pl.ds(sid * RPT, RPT)])


def _scatter(flux, dst_w, zeros):
    fn = pl.kernel(
        _scatter_body,
        out_type=jax.ShapeDtypeStruct((NC, N_ACC, C), jnp.float32),
        mesh=_mesh(),
        scratch_types=[
            pltpu.VMEM((CW, K), jnp.int32),
            pltpu.VMEM((K, C), jnp.float32),
            pltpu.VMEM((K, C), jnp.float32),
            pltpu.VMEM_SHARED((N_ACC, C), jnp.float32),
            pltpu.SemaphoreType.DMA,
            pltpu.SemaphoreType.DMA,
        ],
    )
    return fn(flux, dst_w, zeros)


def _layernorm(h, w, b):
    mu = jnp.mean(h, axis=-1, keepdims=True)
    var = jnp.mean((h - mu) ** 2, axis=-1, keepdims=True)
    return (h - mu) * lax.rsqrt(var + 1e-5) * w + b


def _edge_body(xs_ref, xr_ref, nv0_ref, nv1_ref, fW1_ref, fb1_ref, fW2_ref,
               fb2_ref, flnw_ref, flnb_ref, aW1_ref, ab1_ref, aW2_ref,
               ab2_ref, alnw_ref, alnb_ref, out_ref):
    xs32 = xs_ref[...]
    xr32 = xr_ref[...]
    xs = xs32.astype(jnp.bfloat16)
    xr = xr32.astype(jnp.bfloat16)
    fW1 = fW1_ref[...]
    fW2 = fW2_ref[...]

    def flux_mlp(h):
        h1 = jnp.maximum(
            jnp.dot(h, fW1, preferred_element_type=jnp.float32) + fb1_ref[...],
            0.0)
        h2 = jnp.dot(h1.astype(jnp.bfloat16), fW2,
                     preferred_element_type=jnp.float32) + fb2_ref[...]
        return _layernorm(h2, flnw_ref[...], flnb_ref[...])

    Fs = flux_mlp(xs)
    Fr = flux_mlp(xr)
    m = ((xs32 + xr32) * 0.5).astype(jnp.bfloat16)
    h1 = jnp.maximum(
        jnp.dot(m, aW1_ref[...], preferred_element_type=jnp.float32)
        + ab1_ref[...], 0.0)
    h2 = (jnp.dot(h1.astype(jnp.bfloat16), aW2_ref[...],
                  preferred_element_type=jnp.float32) + ab2_ref[...])
    a = _layernorm(h2, alnw_ref[...], alnb_ref[...])
    fsum = ((Fs[:, :C] + Fr[:, :C]) * nv0_ref[...]
            + (Fs[:, C:] + Fr[:, C:]) * nv1_ref[...])
    out_ref[...] = 0.5 * fsum - 0.5 * a * (xs32 - xr32)


def _edge_tc(xs, xr, nv0, nv1, fW1, fb1, fW2, fb2, flnw, flnb,
             aW1, ab1, aW2, ab2, alnw, alnb):
    nblk = E_PAD // BLK
    edge_spec = pl.BlockSpec((BLK, C), lambda b: (b, 0))
    col_spec = pl.BlockSpec((BLK, 1), lambda b: (b, 0))

    def wspec(arr):
        return pl.BlockSpec(arr.shape, lambda b: (0, 0))

    return pl.pallas_call(
        _edge_body,
        grid=(nblk,),
        in_specs=[edge_spec, edge_spec, col_spec, col_spec,
                  wspec(fW1), wspec(fb1), wspec(fW2), wspec(fb2),
                  wspec(flnw), wspec(flnb), wspec(aW1), wspec(ab1),
                  wspec(aW2), wspec(ab2), wspec(alnw), wspec(alnb)],
        out_specs=edge_spec,
        out_shape=jax.ShapeDtypeStruct((E_PAD, C), jnp.float32),
    )(xs, xr, nv0, nv1, fW1, fb1, fW2, fb2, flnw, flnb,
      aW1, ab1, aW2, ab2, alnw, alnb)


def _combine_body(x_ref, a_ref, b_ref, o_ref):
    o_ref[...] = x_ref[...] + a_ref[...] + b_ref[...]


def _combine(x, a, b):
    spec = pl.BlockSpec((2000, C), lambda i: (i, 0))
    return pl.pallas_call(
        _combine_body,
        grid=(N // 2000,),
        in_specs=[spec, spec, spec],
        out_specs=spec,
        out_shape=jax.ShapeDtypeStruct((N, C), jnp.float32),
    )(x, a, b)


def kernel(x, e, nvec, edge_index, fW1, fb1, fW2, fb2, flnw, flnb,
           aW1, ab1, aW2, ab2, alnw, alnb):
    del e
    src = edge_index[0].astype(jnp.int32)
    dst = edge_index[1].astype(jnp.int32)
    pad = E_PAD - E
    pad_idx = jnp.arange(pad, dtype=jnp.int32)
    src_p = jnp.concatenate([src, pad_idx % N])
    dst_p = jnp.concatenate([dst, N + pad_idx % (N_ACC - N)])
    src_w = src_p.reshape(NW, CW, K)
    dst_w = dst_p.reshape(NW, CW, K)
    nv = jnp.concatenate([nvec.astype(jnp.float32),
                          jnp.zeros((pad, 2), jnp.float32)])
    nv0 = nv[:, 0:1]
    nv1 = nv[:, 1:2]

    xs, xr = _gather(x, src_w, dst_w)

    perm = jnp.concatenate([jnp.arange(0, 2 * C, 2), jnp.arange(1, 2 * C, 2)])
    fW2p = fW2[:, perm]
    fb2p = fb2[perm].reshape(1, 2 * C)
    flnwp = flnw[perm].reshape(1, 2 * C)
    flnbp = flnb[perm].reshape(1, 2 * C)

    flux = _edge_tc(
        xs, xr, nv0, nv1,
        fW1.astype(jnp.bfloat16), fb1.reshape(1, C),
        fW2p.astype(jnp.bfloat16), fb2p, flnwp, flnbp,
        aW1.astype(jnp.bfloat16), ab1.reshape(1, C),
        aW2.astype(jnp.bfloat16), ab2.reshape(1, C),
        alnw.reshape(1, C), alnb.reshape(1, C))

    zeros = jnp.zeros((N_ACC, C), jnp.float32)
    parts = _scatter(flux, dst_w, zeros)
    return _combine(x, parts[0, :N], parts[1, :N])


# speedup vs baseline: 1.0065x; 1.0065x over previous
import functools

import jax
import jax.numpy as jnp
from jax import lax
from jax.experimental import pallas as pl
from jax.experimental.pallas import tpu as pltpu
from jax.experimental.pallas import tpu_sc as plsc

C = 128
N = 10000
E = 320000
NC = 2
NS = 16
NW = NC * NS
K = 128
CW = 80
EW = K * CW
E_PAD = NW * EW
N_ACC = 10240
RPT = N_ACC // NS
BLK = 2048


@functools.lru_cache(maxsize=None)
def _mesh():
    return plsc.VectorSubcoreMesh(
        core_axis_name="c", subcore_axis_name="s",
        num_cores=NC, num_subcores=NS)


NBUF = 3


def _gather_body(x_hbm, src_hbm, dst_hbm, xs_hbm, xr_hbm,
                 idx_s, idx_r, *bufs_and_sems):
    rs = bufs_and_sems[0:NBUF]
    rr = bufs_and_sems[NBUF:2 * NBUF]
    ss = bufs_and_sems[2 * NBUF:3 * NBUF]
    sr = bufs_and_sems[3 * NBUF:4 * NBUF]
    cid = lax.axis_index("c")
    sid = lax.axis_index("s")
    wid = sid * NC + cid
    pltpu.sync_copy(src_hbm.at[wid], idx_s)
    pltpu.sync_copy(dst_hbm.at[wid], idx_r)
    base = wid * EW

    def fire(c, b):
        pltpu.async_copy(x_hbm.at[idx_s.at[c]], rs[b], ss[b])
        pltpu.async_copy(x_hbm.at[idx_r.at[c]], rr[b], sr[b])

    def waitpair(b):
        pltpu.make_async_copy(x_hbm.at[idx_s.at[0]], rs[b], ss[b]).wait()
        pltpu.make_async_copy(x_hbm.at[idx_r.at[0]], rr[b], sr[b]).wait()

    def writeout(c, b):
        off = pl.multiple_of(base + c * K, 8)
        pltpu.sync_copy(rs[b], xs_hbm.at[pl.ds(off, K)])
        pltpu.sync_copy(rr[b], xr_hbm.at[pl.ds(off, K)])

    for b in range(NBUF):
        fire(b, b)

    nsteps = (CW - NBUF) // NBUF

    def step(i, carry):
        c0 = NBUF * i
        for b in range(NBUF):
            waitpair(b)
            writeout(c0 + b, b)
            fire(c0 + b + NBUF, b)
        return carry

    lax.fori_loop(0, nsteps, step, 0)
    done = nsteps * NBUF
    for c in range(done, CW):
        b = c % NBUF
        waitpair(b)
        writeout(c, b)
        if c + NBUF < CW:
            fire(c + NBUF, b)


def _gather(xb, src_w, dst_w):
    fn = pl.kernel(
        _gather_body,
        out_type=(jax.ShapeDtypeStruct((E_PAD, C), jnp.float32),
                  jax.ShapeDtypeStruct((E_PAD, C), jnp.float32)),
        mesh=_mesh(),
        scratch_types=(
            [pltpu.VMEM((CW, K), jnp.int32)] * 2
            + [pltpu.VMEM((K, C), jnp.float32)] * (2 * NBUF)
            + [pltpu.SemaphoreType.DMA] * (2 * NBUF)
        ),
    )
    return fn(xb, src_w, dst_w)


def _scatter_body(flux_hbm, dst_hbm, zeros_hbm, parts_hbm,
                  idx_v, fb0, fb1, acc, sl0, sl1):
    cid = lax.axis_index("c")
    sid = lax.axis_index("s")
    wid = sid * NC + cid
    pltpu.sync_copy(zeros_hbm.at[pl.ds(sid * RPT, RPT)],
                    acc.at[pl.ds(sid * RPT, RPT)])
    pltpu.sync_copy(dst_hbm.at[wid], idx_v)
    plsc.subcore_barrier()
    base = wid * EW

    def fire(c, fb, sl):
        off = pl.multiple_of(base + c * K, 8)
        pltpu.async_copy(flux_hbm.at[pl.ds(off, K)], fb, sl)

    def wait(fb, sl):
        pltpu.make_async_copy(flux_hbm.at[pl.ds(0, K)], fb, sl).wait()

    def scat(c, fb):
        pltpu.sync_copy(fb, acc.at[idx_v.at[c]], add=True)

    fire(0, fb0, sl0)
    fire(1, fb1, sl1)

    def step(i, carry):
        c = 2 * i
        wait(fb0, sl0)
        scat(c, fb0)
        fire(c + 2, fb0, sl0)
        wait(fb1, sl1)
        scat(c + 1, fb1)
        fire(c + 3, fb1, sl1)
        return carry

    lax.fori_loop(0, (CW - 2) // 2, step, 0)
    wait(fb0, sl0)
    scat(CW - 2, fb0)
    wait(fb1, sl1)
    scat(CW - 1, fb1)

    plsc.subcore_barrier()
    pltpu.sync_copy(acc.at[pl.ds(sid * RPT, RPT)],
                    parts_hbm.at[cid, pl.ds(sid * RPT, RPT)])


def _scatter(flux, dst_w, zeros):
    fn = pl.kernel(
        _scatter_body,
        out_type=jax.ShapeDtypeStruct((NC, N_ACC, C), jnp.float32),
        mesh=_mesh(),
        scratch_types=[
            pltpu.VMEM((CW, K), jnp.int32),
            pltpu.VMEM((K, C), jnp.float32),
            pltpu.VMEM((K, C), jnp.float32),
            pltpu.VMEM_SHARED((N_ACC, C), jnp.float32),
            pltpu.SemaphoreType.DMA,
            pltpu.SemaphoreType.DMA,
        ],
    )
    return fn(flux, dst_w, zeros)


def _layernorm(h, w, b):
    mu = jnp.mean(h, axis=-1, keepdims=True)
    var = jnp.mean((h - mu) ** 2, axis=-1, keepdims=True)
    return (h - mu) * lax.rsqrt(var + 1e-5) * w + b


def _edge_body(xs_ref, xr_ref, nv0_ref, nv1_ref, fW1_ref, fb1_ref, fW2_ref,
               fb2_ref, flnw_ref, flnb_ref, aW1_ref, ab1_ref, aW2_ref,
               ab2_ref, alnw_ref, alnb_ref, out_ref):
    xs32 = xs_ref[...]
    xr32 = xr_ref[...]
    xs = xs32.astype(jnp.bfloat16)
    xr = xr32.astype(jnp.bfloat16)
    fW1 = fW1_ref[...]
    fW2 = fW2_ref[...]

    def flux_mlp(h):
        h1 = jnp.maximum(
            jnp.dot(h, fW1, preferred_element_type=jnp.float32) + fb1_ref[...],
            0.0)
        h2 = jnp.dot(h1.astype(jnp.bfloat16), fW2,
                     preferred_element_type=jnp.float32) + fb2_ref[...]
        return _layernorm(h2, flnw_ref[...], flnb_ref[...])

    Fs = flux_mlp(xs)
    Fr = flux_mlp(xr)
    m = ((xs32 + xr32) * 0.5).astype(jnp.bfloat16)
    h1 = jnp.maximum(
        jnp.dot(m, aW1_ref[...], preferred_element_type=jnp.float32)
        + ab1_ref[...], 0.0)
    h2 = (jnp.dot(h1.astype(jnp.bfloat16), aW2_ref[...],
                  preferred_element_type=jnp.float32) + ab2_ref[...])
    a = _layernorm(h2, alnw_ref[...], alnb_ref[...])
    fsum = ((Fs[:, :C] + Fr[:, :C]) * nv0_ref[...]
            + (Fs[:, C:] + Fr[:, C:]) * nv1_ref[...])
    out_ref[...] = 0.5 * fsum - 0.5 * a * (xs32 - xr32)


def _edge_tc(xs, xr, nv0, nv1, fW1, fb1, fW2, fb2, flnw, flnb,
             aW1, ab1, aW2, ab2, alnw, alnb):
    nblk = E_PAD // BLK
    edge_spec = pl.BlockSpec((BLK, C), lambda b: (b, 0))
    col_spec = pl.BlockSpec((BLK, 1), lambda b: (b, 0))

    def wspec(arr):
        return pl.BlockSpec(arr.shape, lambda b: (0, 0))

    return pl.pallas_call(
        _edge_body,
        grid=(nblk,),
        in_specs=[edge_spec, edge_spec, col_spec, col_spec,
                  wspec(fW1), wspec(fb1), wspec(fW2), wspec(fb2),
                  wspec(flnw), wspec(flnb), wspec(aW1), wspec(ab1),
                  wspec(aW2), wspec(ab2), wspec(alnw), wspec(alnb)],
        out_specs=edge_spec,
        out_shape=jax.ShapeDtypeStruct((E_PAD, C), jnp.float32),
    )(xs, xr, nv0, nv1, fW1, fb1, fW2, fb2, flnw, flnb,
      aW1, ab1, aW2, ab2, alnw, alnb)


def _combine_body(x_ref, a_ref, b_ref, o_ref):
    o_ref[...] = x_ref[...] + a_ref[0] + b_ref[0]


def _combine(x, parts):
    spec = pl.BlockSpec((2000, C), lambda i: (i, 0))
    pspec0 = pl.BlockSpec((1, 2000, C), lambda i: (0, i, 0))
    pspec1 = pl.BlockSpec((1, 2000, C), lambda i: (1, i, 0))
    return pl.pallas_call(
        _combine_body,
        grid=(N // 2000,),
        in_specs=[spec, pspec0, pspec1],
        out_specs=spec,
        out_shape=jax.ShapeDtypeStruct((N, C), jnp.float32),
    )(x, parts, parts)


def kernel(x, e, nvec, edge_index, fW1, fb1, fW2, fb2, flnw, flnb,
           aW1, ab1, aW2, ab2, alnw, alnb):
    del e
    src = edge_index[0].astype(jnp.int32)
    dst = edge_index[1].astype(jnp.int32)
    pad = E_PAD - E
    pad_idx = jnp.arange(pad, dtype=jnp.int32)
    src_p = jnp.concatenate([src, pad_idx % N])
    dst_p = jnp.concatenate([dst, N + pad_idx % (N_ACC - N)])
    src_w = src_p.reshape(NW, CW, K)
    dst_w = dst_p.reshape(NW, CW, K)
    nv = jnp.concatenate([nvec.astype(jnp.float32),
                          jnp.zeros((pad, 2), jnp.float32)])
    nv0 = nv[:, 0:1]
    nv1 = nv[:, 1:2]

    xs, xr = _gather(x, src_w, dst_w)

    perm = jnp.concatenate([jnp.arange(0, 2 * C, 2), jnp.arange(1, 2 * C, 2)])
    fW2p = fW2[:, perm]
    fb2p = fb2[perm].reshape(1, 2 * C)
    flnwp = flnw[perm].reshape(1, 2 * C)
    flnbp = flnb[perm].reshape(1, 2 * C)

    flux = _edge_tc(
        xs, xr, nv0, nv1,
        fW1.astype(jnp.bfloat16), fb1.reshape(1, C),
        fW2p.astype(jnp.bfloat16), fb2p, flnwp, flnbp,
        aW1.astype(jnp.bfloat16), ab1.reshape(1, C),
        aW2.astype(jnp.bfloat16), ab2.reshape(1, C),
        alnw.reshape(1, C), alnb.reshape(1, C))

    zeros = jnp.zeros((N_ACC, C), jnp.float32)
    parts = _scatter(flux, dst_w, zeros)
    return _combine(x, parts)


# speedup vs baseline: 1.0208x; 1.0142x over previous
import functools

import jax
import jax.numpy as jnp
from jax import lax
from jax.experimental import pallas as pl
from jax.experimental.pallas import tpu as pltpu
from jax.experimental.pallas import tpu_sc as plsc

C = 128
N = 10000
E = 320000
NC = 2
NS = 16
NW = NC * NS
K = 128
CW = 80
EW = K * CW
E_PAD = NW * EW
N_ACC = 10240
RPT = N_ACC // NS
BLK = 4096


@functools.lru_cache(maxsize=None)
def _mesh():
    return plsc.VectorSubcoreMesh(
        core_axis_name="c", subcore_axis_name="s",
        num_cores=NC, num_subcores=NS)


NBUF = 3


def _gather_body(x_hbm, src_hbm, dst_hbm, xs_hbm, xr_hbm,
                 idx_s, idx_r, *bufs_and_sems):
    rs = bufs_and_sems[0:NBUF]
    rr = bufs_and_sems[NBUF:2 * NBUF]
    ss = bufs_and_sems[2 * NBUF:3 * NBUF]
    sr = bufs_and_sems[3 * NBUF:4 * NBUF]
    cid = lax.axis_index("c")
    sid = lax.axis_index("s")
    wid = sid * NC + cid
    pltpu.sync_copy(src_hbm.at[wid], idx_s)
    pltpu.sync_copy(dst_hbm.at[wid], idx_r)
    base = wid * EW

    def fire(c, b):
        pltpu.async_copy(x_hbm.at[idx_s.at[c]], rs[b], ss[b])
        pltpu.async_copy(x_hbm.at[idx_r.at[c]], rr[b], sr[b])

    def waitpair(b):
        pltpu.make_async_copy(x_hbm.at[idx_s.at[0]], rs[b], ss[b]).wait()
        pltpu.make_async_copy(x_hbm.at[idx_r.at[0]], rr[b], sr[b]).wait()

    def writeout(c, b):
        off = pl.multiple_of(base + c * K, 8)
        pltpu.sync_copy(rs[b], xs_hbm.at[pl.ds(off, K)])
        pltpu.sync_copy(rr[b], xr_hbm.at[pl.ds(off, K)])

    for b in range(NBUF):
        fire(b, b)

    nsteps = (CW - NBUF) // NBUF

    def step(i, carry):
        c0 = NBUF * i
        for b in range(NBUF):
            waitpair(b)
            writeout(c0 + b, b)
            fire(c0 + b + NBUF, b)
        return carry

    lax.fori_loop(0, nsteps, step, 0)
    done = nsteps * NBUF
    for c in range(done, CW):
        b = c % NBUF
        waitpair(b)
        writeout(c, b)
        if c + NBUF < CW:
            fire(c + NBUF, b)


def _gather(xb, src_w, dst_w):
    fn = pl.kernel(
        _gather_body,
        out_type=(jax.ShapeDtypeStruct((E_PAD, C), jnp.float32),
                  jax.ShapeDtypeStruct((E_PAD, C), jnp.float32)),
        mesh=_mesh(),
        scratch_types=(
            [pltpu.VMEM((CW, K), jnp.int32)] * 2
            + [pltpu.VMEM((K, C), jnp.float32)] * (2 * NBUF)
            + [pltpu.SemaphoreType.DMA] * (2 * NBUF)
        ),
    )
    return fn(xb, src_w, dst_w)


def _scatter_body(flux_hbm, dst_hbm, zeros_hbm, parts_hbm,
                  idx_v, fb0, fb1, acc, sl0, sl1):
    cid = lax.axis_index("c")
    sid = lax.axis_index("s")
    wid = sid * NC + cid
    pltpu.sync_copy(zeros_hbm.at[pl.ds(sid * RPT, RPT)],
                    acc.at[pl.ds(sid * RPT, RPT)])
    pltpu.sync_copy(dst_hbm.at[wid], idx_v)
    plsc.subcore_barrier()
    base = wid * EW

    def fire(c, fb, sl):
        off = pl.multiple_of(base + c * K, 8)
        pltpu.async_copy(flux_hbm.at[pl.ds(off, K)], fb, sl)

    def wait(fb, sl):
        pltpu.make_async_copy(flux_hbm.at[pl.ds(0, K)], fb, sl).wait()

    def scat(c, fb):
        pltpu.sync_copy(fb, acc.at[idx_v.at[c]], add=True)

    fire(0, fb0, sl0)
    fire(1, fb1, sl1)

    def step(i, carry):
        c = 2 * i
        wait(fb0, sl0)
        scat(c, fb0)
        fire(c + 2, fb0, sl0)
        wait(fb1, sl1)
        scat(c + 1, fb1)
        fire(c + 3, fb1, sl1)
        return carry

    lax.fori_loop(0, (CW - 2) // 2, step, 0)
    wait(fb0, sl0)
    scat(CW - 2, fb0)
    wait(fb1, sl1)
    scat(CW - 1, fb1)

    plsc.subcore_barrier()
    pltpu.sync_copy(acc.at[pl.ds(sid * RPT, RPT)],
                    parts_hbm.at[cid, pl.ds(sid * RPT, RPT)])


def _scatter(flux, dst_w, zeros):
    fn = pl.kernel(
        _scatter_body,
        out_type=jax.ShapeDtypeStruct((NC, N_ACC, C), jnp.float32),
        mesh=_mesh(),
        scratch_types=[
            pltpu.VMEM((CW, K), jnp.int32),
            pltpu.VMEM((K, C), jnp.float32),
            pltpu.VMEM((K, C), jnp.float32),
            pltpu.VMEM_SHARED((N_ACC, C), jnp.float32),
            pltpu.SemaphoreType.DMA,
            pltpu.SemaphoreType.DMA,
        ],
    )
    return fn(flux, dst_w, zeros)


def _layernorm(h, w, b):
    mu = jnp.mean(h, axis=-1, keepdims=True)
    var = jnp.mean((h - mu) ** 2, axis=-1, keepdims=True)
    return (h - mu) * lax.rsqrt(var + 1e-5) * w + b


def _edge_body(xs_ref, xr_ref, nv0_ref, nv1_ref, fW1_ref, fb1_ref, fW2_ref,
               fb2_ref, flnw_ref, flnb_ref, aW1_ref, ab1_ref, aW2_ref,
               ab2_ref, alnw_ref, alnb_ref, out_ref):
    xs32 = xs_ref[...]
    xr32 = xr_ref[...]
    xs = xs32.astype(jnp.bfloat16)
    xr = xr32.astype(jnp.bfloat16)
    fW1 = fW1_ref[...]
    fW2 = fW2_ref[...]

    def flux_mlp(h):
        h1 = jnp.maximum(
            jnp.dot(h, fW1, preferred_element_type=jnp.float32) + fb1_ref[...],
            0.0)
        h2 = jnp.dot(h1.astype(jnp.bfloat16), fW2,
                     preferred_element_type=jnp.float32) + fb2_ref[...]
        return _layernorm(h2, flnw_ref[...], flnb_ref[...])

    Fs = flux_mlp(xs)
    Fr = flux_mlp(xr)
    m = ((xs32 + xr32) * 0.5).astype(jnp.bfloat16)
    h1 = jnp.maximum(
        jnp.dot(m, aW1_ref[...], preferred_element_type=jnp.float32)
        + ab1_ref[...], 0.0)
    h2 = (jnp.dot(h1.astype(jnp.bfloat16), aW2_ref[...],
                  preferred_element_type=jnp.float32) + ab2_ref[...])
    a = _layernorm(h2, alnw_ref[...], alnb_ref[...])
    fsum = ((Fs[:, :C] + Fr[:, :C]) * nv0_ref[...]
            + (Fs[:, C:] + Fr[:, C:]) * nv1_ref[...])
    out_ref[...] = 0.5 * fsum - 0.5 * a * (xs32 - xr32)


def _edge_tc(xs, xr, nv0, nv1, fW1, fb1, fW2, fb2, flnw, flnb,
             aW1, ab1, aW2, ab2, alnw, alnb):
    nblk = E_PAD // BLK
    edge_spec = pl.BlockSpec((BLK, C), lambda b: (b, 0))
    col_spec = pl.BlockSpec((BLK, 1), lambda b: (b, 0))

    def wspec(arr):
        return pl.BlockSpec(arr.shape, lambda b: (0, 0))

    return pl.pallas_call(
        _edge_body,
        grid=(nblk,),
        in_specs=[edge_spec, edge_spec, col_spec, col_spec,
                  wspec(fW1), wspec(fb1), wspec(fW2), wspec(fb2),
                  wspec(flnw), wspec(flnb), wspec(aW1), wspec(ab1),
                  wspec(aW2), wspec(ab2), wspec(alnw), wspec(alnb)],
        out_specs=edge_spec,
        out_shape=jax.ShapeDtypeStruct((E_PAD, C), jnp.float32),
    )(xs, xr, nv0, nv1, fW1, fb1, fW2, fb2, flnw, flnb,
      aW1, ab1, aW2, ab2, alnw, alnb)


def _combine_body(x_ref, a_ref, b_ref, o_ref):
    o_ref[...] = x_ref[...] + a_ref[0] + b_ref[0]


def _combine(x, parts):
    spec = pl.BlockSpec((2000, C), lambda i: (i, 0))
    pspec0 = pl.BlockSpec((1, 2000, C), lambda i: (0, i, 0))
    pspec1 = pl.BlockSpec((1, 2000, C), lambda i: (1, i, 0))
    return pl.pallas_call(
        _combine_body,
        grid=(N // 2000,),
        in_specs=[spec, pspec0, pspec1],
        out_specs=spec,
        out_shape=jax.ShapeDtypeStruct((N, C), jnp.float32),
    )(x, parts, parts)


def kernel(x, e, nvec, edge_index, fW1, fb1, fW2, fb2, flnw, flnb,
           aW1, ab1, aW2, ab2, alnw, alnb):
    del e
    src = edge_index[0].astype(jnp.int32)
    dst = edge_index[1].astype(jnp.int32)
    pad = E_PAD - E
    pad_idx = jnp.arange(pad, dtype=jnp.int32)
    src_p = jnp.concatenate([src, pad_idx % N])
    dst_p = jnp.concatenate([dst, N + pad_idx % (N_ACC - N)])
    src_w = src_p.reshape(NW, CW, K)
    dst_w = dst_p.reshape(NW, CW, K)
    nv = jnp.concatenate([nvec.astype(jnp.float32),
                          jnp.zeros((pad, 2), jnp.float32)])
    nv0 = nv[:, 0:1]
    nv1 = nv[:, 1:2]

    xs, xr = _gather(x, src_w, dst_w)

    perm = jnp.concatenate([jnp.arange(0, 2 * C, 2), jnp.arange(1, 2 * C, 2)])
    fW2p = fW2[:, perm]
    fb2p = fb2[perm].reshape(1, 2 * C)
    flnwp = flnw[perm].reshape(1, 2 * C)
    flnbp = flnb[perm].reshape(1, 2 * C)

    flux = _edge_tc(
        xs, xr, nv0, nv1,
        fW1.astype(jnp.bfloat16), fb1.reshape(1, C),
        fW2p.astype(jnp.bfloat16), fb2p, flnwp, flnbp,
        aW1.astype(jnp.bfloat16), ab1.reshape(1, C),
        aW2.astype(jnp.bfloat16), ab2.reshape(1, C),
        alnw.reshape(1, C), alnb.reshape(1, C))

    zeros = jnp.zeros((N_ACC, C), jnp.float32)
    parts = _scatter(flux, dst_w, zeros)
    return _combine(x, parts)
